# baseline (speedup 1.0000x reference)
import functools

import jax
import jax.numpy as jnp
from jax import lax
from jax.experimental import pallas as pl
from jax.experimental.pallas import tpu as pltpu
from jax.experimental.pallas import tpu_sc as plsc

_MARGIN = 0.3
_EPS = 1e-6
_B = 256
_D = 2048
_L = 16
_NC = 2
_NS = 16
_RPW = _B // _L


def _threefry2x32(k0, k1, c0, c1):
    import numpy as np

    def rotl(x, r):
        return ((x << np.uint32(r)) | (x >> np.uint32(32 - r))).astype(np.uint32)

    ks0, ks1 = np.uint32(k0), np.uint32(k1)
    ks2 = np.uint32(ks0 ^ ks1 ^ np.uint32(0x1BD11BDA))
    x0 = (np.asarray(c0, np.uint32) + ks0).astype(np.uint32)
    x1 = (np.asarray(c1, np.uint32) + ks1).astype(np.uint32)
    rot = [13, 15, 26, 6, 17, 29, 16, 24]
    inject = [(ks1, ks2), (ks2, ks0), (ks0, ks1), (ks1, ks2), (ks2, ks0)]
    for block in range(5):
        for r in (rot[:4] if block % 2 == 0 else rot[4:]):
            x0 = (x0 + x1).astype(np.uint32)
            x1 = rotl(x1, r)
            x1 = (x1 ^ x0).astype(np.uint32)
        a, b = inject[block]
        x0 = (x0 + a).astype(np.uint32)
        x1 = (x1 + b + np.uint32(block + 1)).astype(np.uint32)
    return x0, x1


def _selection_constants():
    import numpy as np

    def bits(kpair, n):
        x0, x1 = _threefry2x32(kpair[0], kpair[1],
                               np.zeros(n, np.uint32), np.arange(n, dtype=np.uint32))
        return (x0 ^ x1).astype(np.uint32)

    def uniform(kpair, n):
        b = bits(kpair, n)
        return (((b >> np.uint32(9)) | np.uint32(0x3F800000))
                .view(np.float32) - np.float32(1.0))

    s0, s1 = _threefry2x32(0, 42, np.zeros(3, np.uint32),
                           np.arange(3, dtype=np.uint32))
    gp = uniform((s0[0], s1[0]), _B * _B).reshape(_B, _B)
    gn = uniform((s0[1], s1[1]), _B * _B).reshape(_B, _B)
    fb = np.array([
        49, 93, 107, 176, 77, 114, 51, 105, 130, 195, 217, 87, 120, 11, 158,
        226, 12, 194, 253, 69, 5, 212, 247, 10, 133, 85, 245, 148, 151, 21,
        85, 102, 134, 124, 40, 8, 221, 89, 168, 108, 46, 154, 166, 72, 79,
        247, 19, 10, 114, 97, 15, 77, 12, 147, 251, 16, 62, 79, 122, 230,
        220, 73, 255, 234, 10, 7, 68, 201, 10, 163, 63, 99, 86, 238, 223,
        225, 123, 53, 46, 45, 17, 243, 96, 79, 210, 106, 69, 109, 158, 13,
        165, 189, 155, 144, 61, 196, 34, 114, 177, 153, 81, 100, 47, 114,
        19, 27, 193, 146, 144, 255, 55, 68, 208, 64, 149, 244, 2, 101, 151,
        122, 40, 107, 24, 8, 127, 37, 24, 18, 27, 221, 33, 238, 66, 162,
        123, 151, 243, 149, 67, 177, 201, 202, 34, 250, 251, 7, 154, 16,
        222, 33, 75, 28, 120, 33, 232, 157, 170, 82, 124, 216, 91, 239, 147,
        162, 29, 60, 239, 153, 41, 106, 188, 95, 157, 76, 181, 70, 114, 71,
        216, 227, 9, 186, 77, 246, 94, 27, 111, 167, 100, 59, 134, 203, 246,
        241, 223, 60, 189, 156, 212, 129, 33, 111, 228, 52, 117, 145, 180,
        135, 69, 31, 101, 15, 250, 169, 151, 41, 231, 83, 93, 50, 9, 161,
        238, 221, 224, 3, 65, 155, 5, 194, 84, 70, 221, 114, 10, 141, 161,
        44, 10, 79, 119, 91, 181, 181, 59, 237, 86, 17, 51, 247, 139, 222,
        214, 6, 4, 3], dtype=np.int32)
    return gp, gn, fb


_GP, _GN, _FB = _selection_constants()


def _sc_select_body(labels_hbm, g_hbm, fb_hbm, pos_hbm, neg_hbm,
                    labels_v, g_v, fb_v, res_v):
    wid = lax.axis_index("s") * _NC + lax.axis_index("c")
    is_pos = wid < _NS
    row0 = jnp.where(is_pos, wid, wid - _NS) * _RPW

    pltpu.sync_copy(labels_hbm, labels_v)
    pltpu.sync_copy(g_hbm.at[wid], g_v)
    pltpu.sync_copy(fb_hbm.at[pl.ds(row0, _RPW)], fb_v)

    labm = labels_v[pl.ds(row0, _L)]
    flip = jnp.full((_L,), (wid >= _NS).astype(jnp.int32))

    def outer(jo, carry):
        bestv, besti = carry
        for ji in range(_L):
            j = jo * _L + ji
            lj_v = plsc.load_gather(labels_v, [jnp.full((_L,), j, jnp.int32)])
            same = (labm == lj_v).astype(jnp.int32)
            g = g_v[j]
            mg = jnp.where((same ^ flip) != 0, g, -1.0)
            upd = mg > bestv
            bestv = jnp.where(upd, mg, bestv)
            besti = jnp.where(upd, jnp.full((_L,), j, jnp.int32), besti)
        return bestv, besti

    bestv, besti = lax.fori_loop(
        0, _B // _L, outer,
        (jnp.full((_L,), -2.0, jnp.float32), jnp.zeros((_L,), jnp.int32)))

    res = jnp.where(bestv > -1.0, besti, fb_v[...])
    res_v[...] = res

    @pl.when(is_pos)
    def _():
        pltpu.sync_copy(res_v, pos_hbm.at[pl.ds(row0, _RPW)])

    @pl.when(jnp.logical_not(is_pos))
    def _():
        pltpu.sync_copy(res_v, neg_hbm.at[pl.ds(row0, _RPW)])


@functools.cache
def _sc_select():
    mesh = plsc.VectorSubcoreMesh(core_axis_name="c", subcore_axis_name="s")
    return pl.kernel(
        _sc_select_body,
        mesh=mesh,
        compiler_params=pltpu.CompilerParams(needs_layout_passes=False),
        out_type=[jax.ShapeDtypeStruct((_B,), jnp.int32),
                  jax.ShapeDtypeStruct((_B,), jnp.int32)],
        scratch_types=[pltpu.VMEM((_B,), jnp.int32),
                       pltpu.VMEM((_B, _L), jnp.float32),
                       pltpu.VMEM((_RPW,), jnp.int32),
                       pltpu.VMEM((_L,), jnp.int32)],
    )


def _worker_g_layout():
    import numpy as np
    g = np.empty((2 * _NS, _B, _L), np.float32)
    for w in range(_NS):
        g[w] = _GP[w * _L:(w + 1) * _L, :].T
        g[_NS + w] = _GN[w * _L:(w + 1) * _L, :].T
    return g


_GW = _worker_g_layout()


def _dense_kernel(f_ref, pidx_ref, nidx_ref, out_ref):
    f = f_ref[...]
    iota_j = lax.broadcasted_iota(jnp.int32, (_B, _B), 1)
    P = (iota_j == pidx_ref[...]).astype(jnp.float32)
    N = (iota_j == nidx_ref[...]).astype(jnp.float32)

    gram = lax.dot_general(f, f, (((1,), (1,)), ((), ())),
                           preferred_element_type=jnp.float32)
    sq = jnp.sum(f * f, axis=1, keepdims=True)
    rs = jnp.sum(f, axis=1, keepdims=True)

    dotp = lax.dot_general(P, gram, (((1,), (0,)), ((), ())),
                           preferred_element_type=jnp.float32)
    dotn = lax.dot_general(N, gram, (((1,), (0,)), ((), ())),
                           preferred_element_type=jnp.float32)
    aux = jnp.concatenate([sq, rs], axis=1)
    auxp = lax.dot_general(P, aux, (((1,), (0,)), ((), ())),
                           preferred_element_type=jnp.float32)
    auxn = lax.dot_general(N, aux, (((1,), (0,)), ((), ())),
                           preferred_element_type=jnp.float32)

    sq_row = jnp.transpose(sq)
    rs_row = jnp.transpose(rs)
    const = float(_D) * _EPS * _EPS

    sqp = sq_row + auxp[:, 0:1] - 2.0 * dotp \
        + 2.0 * _EPS * (rs_row - auxp[:, 1:2]) + const
    sqn = sq_row + auxn[:, 0:1] - 2.0 * dotn \
        + 2.0 * _EPS * (rs_row - auxn[:, 1:2]) + const
    dp = jnp.sqrt(jnp.maximum(sqp, 1e-12))
    dn = jnp.sqrt(jnp.maximum(sqn, 1e-12))
    loss = jnp.sum(jnp.maximum(dp - dn + _MARGIN, 0.0),
                   keepdims=True) * (1.0 / (_B * _B))
    out_ref[...] = loss


@jax.jit
def kernel(features, labels):
    gw = jnp.asarray(_GW)
    fb = jnp.asarray(_FB)

    pos_idx, neg_idx = _sc_select()(labels, gw, fb)

    out = pl.pallas_call(
        _dense_kernel,
        out_shape=jax.ShapeDtypeStruct((1, 1), jnp.float32),
    )(features, pos_idx.reshape(_B, 1), neg_idx.reshape(_B, 1))
    return out.reshape(())


# speedup vs baseline: 1.0979x; 1.0979x over previous
import functools

import jax
import jax.numpy as jnp
from jax import lax
from jax.experimental import pallas as pl
from jax.experimental.pallas import tpu as pltpu
from jax.experimental.pallas import tpu_sc as plsc

_MARGIN = 0.3
_EPS = 1e-6
_B = 256
_D = 2048
_L = 16
_NC = 2
_NS = 16
_RPW = _B // _L


def _threefry2x32(k0, k1, c0, c1):
    import numpy as np

    def rotl(x, r):
        return ((x << np.uint32(r)) | (x >> np.uint32(32 - r))).astype(np.uint32)

    ks0, ks1 = np.uint32(k0), np.uint32(k1)
    ks2 = np.uint32(ks0 ^ ks1 ^ np.uint32(0x1BD11BDA))
    x0 = (np.asarray(c0, np.uint32) + ks0).astype(np.uint32)
    x1 = (np.asarray(c1, np.uint32) + ks1).astype(np.uint32)
    rot = [13, 15, 26, 6, 17, 29, 16, 24]
    inject = [(ks1, ks2), (ks2, ks0), (ks0, ks1), (ks1, ks2), (ks2, ks0)]
    for block in range(5):
        for r in (rot[:4] if block % 2 == 0 else rot[4:]):
            x0 = (x0 + x1).astype(np.uint32)
            x1 = rotl(x1, r)
            x1 = (x1 ^ x0).astype(np.uint32)
        a, b = inject[block]
        x0 = (x0 + a).astype(np.uint32)
        x1 = (x1 + b + np.uint32(block + 1)).astype(np.uint32)
    return x0, x1


def _selection_constants():
    import numpy as np

    def bits(kpair, n):
        x0, x1 = _threefry2x32(kpair[0], kpair[1],
                               np.zeros(n, np.uint32), np.arange(n, dtype=np.uint32))
        return (x0 ^ x1).astype(np.uint32)

    def uniform(kpair, n):
        b = bits(kpair, n)
        return (((b >> np.uint32(9)) | np.uint32(0x3F800000))
                .view(np.float32) - np.float32(1.0))

    s0, s1 = _threefry2x32(0, 42, np.zeros(3, np.uint32),
                           np.arange(3, dtype=np.uint32))
    gp = uniform((s0[0], s1[0]), _B * _B).reshape(_B, _B)
    gn = uniform((s0[1], s1[1]), _B * _B).reshape(_B, _B)
    fb = np.array([
        49, 93, 107, 176, 77, 114, 51, 105, 130, 195, 217, 87, 120, 11, 158,
        226, 12, 194, 253, 69, 5, 212, 247, 10, 133, 85, 245, 148, 151, 21,
        85, 102, 134, 124, 40, 8, 221, 89, 168, 108, 46, 154, 166, 72, 79,
        247, 19, 10, 114, 97, 15, 77, 12, 147, 251, 16, 62, 79, 122, 230,
        220, 73, 255, 234, 10, 7, 68, 201, 10, 163, 63, 99, 86, 238, 223,
        225, 123, 53, 46, 45, 17, 243, 96, 79, 210, 106, 69, 109, 158, 13,
        165, 189, 155, 144, 61, 196, 34, 114, 177, 153, 81, 100, 47, 114,
        19, 27, 193, 146, 144, 255, 55, 68, 208, 64, 149, 244, 2, 101, 151,
        122, 40, 107, 24, 8, 127, 37, 24, 18, 27, 221, 33, 238, 66, 162,
        123, 151, 243, 149, 67, 177, 201, 202, 34, 250, 251, 7, 154, 16,
        222, 33, 75, 28, 120, 33, 232, 157, 170, 82, 124, 216, 91, 239, 147,
        162, 29, 60, 239, 153, 41, 106, 188, 95, 157, 76, 181, 70, 114, 71,
        216, 227, 9, 186, 77, 246, 94, 27, 111, 167, 100, 59, 134, 203, 246,
        241, 223, 60, 189, 156, 212, 129, 33, 111, 228, 52, 117, 145, 180,
        135, 69, 31, 101, 15, 250, 169, 151, 41, 231, 83, 93, 50, 9, 161,
        238, 221, 224, 3, 65, 155, 5, 194, 84, 70, 221, 114, 10, 141, 161,
        44, 10, 79, 119, 91, 181, 181, 59, 237, 86, 17, 51, 247, 139, 222,
        214, 6, 4, 3], dtype=np.int32)
    return gp, gn, fb


_GP, _GN, _FB = _selection_constants()


def _sc_select_body(labels_hbm, g_hbm, fb_hbm, pos_hbm, neg_hbm,
                    labels_v, g_v, fb_v, res_v):
    wid = lax.axis_index("s") * _NC + lax.axis_index("c")
    is_pos = wid < _NS
    row0 = jnp.where(is_pos, wid, wid - _NS) * _RPW

    pltpu.sync_copy(labels_hbm, labels_v)
    pltpu.sync_copy(g_hbm.at[wid], g_v)
    pltpu.sync_copy(fb_hbm.at[pl.ds(row0, _RPW)], fb_v)

    flip = jnp.full((_L,), (wid >= _NS).astype(jnp.int32))
    iota = lax.iota(jnp.int32, _L)
    mvec = jnp.full((_L,), -2.0, jnp.float32)
    ivec = jnp.zeros((_L,), jnp.int32)

    for r in range(_RPW):
        myl = plsc.load_gather(
            labels_v, [jnp.full((_L,), row0 + r, jnp.int32)])

        def chunk(jo, carry):
            bestv, besti = carry
            lab_c = labels_v[pl.ds(jo * _L, _L)]
            g_c = g_v[pl.ds(jo * _L + r * _B, _L)]
            same = (lab_c == myl).astype(jnp.int32)
            mg = jnp.where((same ^ flip) != 0, g_c, -1.0)
            upd = mg > bestv
            bestv = jnp.where(upd, mg, bestv)
            besti = jnp.where(upd, jo * _L + iota, besti)
            return bestv, besti

        bestv, besti = lax.fori_loop(
            0, _B // _L, chunk,
            (jnp.full((_L,), -2.0, jnp.float32), jnp.zeros((_L,), jnp.int32)))

        m = jnp.max(bestv)
        idx = jnp.min(jnp.where(bestv == m, besti, jnp.int32(1 << 30)))
        sel = iota == r
        mvec = jnp.where(sel, jnp.full((_L,), m), mvec)
        ivec = jnp.where(sel, jnp.full((_L,), idx), ivec)

    res = jnp.where(mvec > -1.0, ivec, fb_v[...])
    res_v[...] = res

    @pl.when(is_pos)
    def _():
        pltpu.sync_copy(res_v, pos_hbm.at[pl.ds(row0, _RPW)])

    @pl.when(jnp.logical_not(is_pos))
    def _():
        pltpu.sync_copy(res_v, neg_hbm.at[pl.ds(row0, _RPW)])


@functools.cache
def _sc_select():
    mesh = plsc.VectorSubcoreMesh(core_axis_name="c", subcore_axis_name="s")
    return pl.kernel(
        _sc_select_body,
        mesh=mesh,
        compiler_params=pltpu.CompilerParams(needs_layout_passes=False),
        out_type=[jax.ShapeDtypeStruct((_B,), jnp.int32),
                  jax.ShapeDtypeStruct((_B,), jnp.int32)],
        scratch_types=[pltpu.VMEM((_B,), jnp.int32),
                       pltpu.VMEM((_RPW * _B,), jnp.float32),
                       pltpu.VMEM((_RPW,), jnp.int32),
                       pltpu.VMEM((_L,), jnp.int32)],
    )


def _worker_g_layout():
    import numpy as np
    g = np.empty((2 * _NS, _RPW * _B), np.float32)
    for w in range(_NS):
        g[w] = _GP[w * _RPW:(w + 1) * _RPW, :].reshape(-1)
        g[_NS + w] = _GN[w * _RPW:(w + 1) * _RPW, :].reshape(-1)
    return g


_GW = _worker_g_layout()


def _dense_kernel(f_ref, pidx_ref, nidx_ref, out_ref):
    f = f_ref[...]
    iota_j = lax.broadcasted_iota(jnp.int32, (_B, _B), 1)
    P = (iota_j == pidx_ref[...]).astype(jnp.float32)
    N = (iota_j == nidx_ref[...]).astype(jnp.float32)

    gram = lax.dot_general(f, f, (((1,), (1,)), ((), ())),
                           preferred_element_type=jnp.float32)
    sq = jnp.sum(f * f, axis=1, keepdims=True)
    rs = jnp.sum(f, axis=1, keepdims=True)

    dotp = lax.dot_general(P, gram, (((1,), (0,)), ((), ())),
                           preferred_element_type=jnp.float32)
    dotn = lax.dot_general(N, gram, (((1,), (0,)), ((), ())),
                           preferred_element_type=jnp.float32)
    aux = jnp.concatenate([sq, rs], axis=1)
    auxp = lax.dot_general(P, aux, (((1,), (0,)), ((), ())),
                           preferred_element_type=jnp.float32)
    auxn = lax.dot_general(N, aux, (((1,), (0,)), ((), ())),
                           preferred_element_type=jnp.float32)

    sq_row = jnp.transpose(sq)
    rs_row = jnp.transpose(rs)
    const = float(_D) * _EPS * _EPS

    sqp = sq_row + auxp[:, 0:1] - 2.0 * dotp \
        + 2.0 * _EPS * (rs_row - auxp[:, 1:2]) + const
    sqn = sq_row + auxn[:, 0:1] - 2.0 * dotn \
        + 2.0 * _EPS * (rs_row - auxn[:, 1:2]) + const
    dp = jnp.sqrt(jnp.maximum(sqp, 1e-12))
    dn = jnp.sqrt(jnp.maximum(sqn, 1e-12))
    loss = jnp.sum(jnp.maximum(dp - dn + _MARGIN, 0.0),
                   keepdims=True) * (1.0 / (_B * _B))
    out_ref[...] = loss


@jax.jit
def kernel(features, labels):
    gw = jnp.asarray(_GW)
    fb = jnp.asarray(_FB)

    pos_idx, neg_idx = _sc_select()(labels, gw, fb)

    out = pl.pallas_call(
        _dense_kernel,
        out_shape=jax.ShapeDtypeStruct((1, 1), jnp.float32),
    )(features, pos_idx.reshape(_B, 1), neg_idx.reshape(_B, 1))
    return out.reshape(())


# speedup vs baseline: 1.1199x; 1.0200x over previous
import functools

import jax
import jax.numpy as jnp
from jax import lax
from jax.experimental import pallas as pl
from jax.experimental.pallas import tpu as pltpu
from jax.experimental.pallas import tpu_sc as plsc

_MARGIN = 0.3
_EPS = 1e-6
_B = 256
_D = 2048
_L = 16
_NC = 2
_NS = 16
_RPW = _B // _L


def _threefry2x32(k0, k1, c0, c1):
    import numpy as np

    def rotl(x, r):
        return ((x << np.uint32(r)) | (x >> np.uint32(32 - r))).astype(np.uint32)

    ks0, ks1 = np.uint32(k0), np.uint32(k1)
    ks2 = np.uint32(ks0 ^ ks1 ^ np.uint32(0x1BD11BDA))
    x0 = (np.asarray(c0, np.uint32) + ks0).astype(np.uint32)
    x1 = (np.asarray(c1, np.uint32) + ks1).astype(np.uint32)
    rot = [13, 15, 26, 6, 17, 29, 16, 24]
    inject = [(ks1, ks2), (ks2, ks0), (ks0, ks1), (ks1, ks2), (ks2, ks0)]
    for block in range(5):
        for r in (rot[:4] if block % 2 == 0 else rot[4:]):
            x0 = (x0 + x1).astype(np.uint32)
            x1 = rotl(x1, r)
            x1 = (x1 ^ x0).astype(np.uint32)
        a, b = inject[block]
        x0 = (x0 + a).astype(np.uint32)
        x1 = (x1 + b + np.uint32(block + 1)).astype(np.uint32)
    return x0, x1


def _selection_constants():
    import numpy as np

    def bits(kpair, n):
        x0, x1 = _threefry2x32(kpair[0], kpair[1],
                               np.zeros(n, np.uint32), np.arange(n, dtype=np.uint32))
        return (x0 ^ x1).astype(np.uint32)

    def uniform(kpair, n):
        b = bits(kpair, n)
        return (((b >> np.uint32(9)) | np.uint32(0x3F800000))
                .view(np.float32) - np.float32(1.0))

    s0, s1 = _threefry2x32(0, 42, np.zeros(3, np.uint32),
                           np.arange(3, dtype=np.uint32))
    gp = uniform((s0[0], s1[0]), _B * _B).reshape(_B, _B)
    gn = uniform((s0[1], s1[1]), _B * _B).reshape(_B, _B)
    fb = np.array([
        49, 93, 107, 176, 77, 114, 51, 105, 130, 195, 217, 87, 120, 11, 158,
        226, 12, 194, 253, 69, 5, 212, 247, 10, 133, 85, 245, 148, 151, 21,
        85, 102, 134, 124, 40, 8, 221, 89, 168, 108, 46, 154, 166, 72, 79,
        247, 19, 10, 114, 97, 15, 77, 12, 147, 251, 16, 62, 79, 122, 230,
        220, 73, 255, 234, 10, 7, 68, 201, 10, 163, 63, 99, 86, 238, 223,
        225, 123, 53, 46, 45, 17, 243, 96, 79, 210, 106, 69, 109, 158, 13,
        165, 189, 155, 144, 61, 196, 34, 114, 177, 153, 81, 100, 47, 114,
        19, 27, 193, 146, 144, 255, 55, 68, 208, 64, 149, 244, 2, 101, 151,
        122, 40, 107, 24, 8, 127, 37, 24, 18, 27, 221, 33, 238, 66, 162,
        123, 151, 243, 149, 67, 177, 201, 202, 34, 250, 251, 7, 154, 16,
        222, 33, 75, 28, 120, 33, 232, 157, 170, 82, 124, 216, 91, 239, 147,
        162, 29, 60, 239, 153, 41, 106, 188, 95, 157, 76, 181, 70, 114, 71,
        216, 227, 9, 186, 77, 246, 94, 27, 111, 167, 100, 59, 134, 203, 246,
        241, 223, 60, 189, 156, 212, 129, 33, 111, 228, 52, 117, 145, 180,
        135, 69, 31, 101, 15, 250, 169, 151, 41, 231, 83, 93, 50, 9, 161,
        238, 221, 224, 3, 65, 155, 5, 194, 84, 70, 221, 114, 10, 141, 161,
        44, 10, 79, 119, 91, 181, 181, 59, 237, 86, 17, 51, 247, 139, 222,
        214, 6, 4, 3], dtype=np.int32)
    return gp, gn, fb


_GP, _GN, _FB = _selection_constants()


def _sc_select_body(labels_hbm, g_hbm, fb_hbm, pos_hbm, neg_hbm,
                    labels_v, g_v, fb_v, res_v):
    wid = lax.axis_index("s") * _NC + lax.axis_index("c")
    is_pos = wid < _NS
    row0 = jnp.where(is_pos, wid, wid - _NS) * _RPW

    pltpu.sync_copy(labels_hbm, labels_v)
    pltpu.sync_copy(g_hbm.at[wid], g_v)
    pltpu.sync_copy(fb_hbm.at[pl.ds(row0, _RPW)], fb_v)

    flip = jnp.full((_L,), (wid >= _NS).astype(jnp.int32))
    iota = lax.iota(jnp.int32, _L)
    mvec = jnp.full((_L,), -2.0, jnp.float32)
    ivec = jnp.zeros((_L,), jnp.int32)

    for r in range(_RPW):
        myl = plsc.load_gather(
            labels_v, [jnp.full((_L,), row0 + r, jnp.int32)])

        def chunk(jo, carry):
            bestv, besti = carry
            lab_c = labels_v[pl.ds(jo * _L, _L)]
            g_c = g_v[pl.ds(jo * _L + r * _B, _L)]
            same = (lab_c == myl).astype(jnp.int32)
            mg = jnp.where((same ^ flip) != 0, g_c, -1.0)
            upd = mg > bestv
            bestv = jnp.where(upd, mg, bestv)
            besti = jnp.where(upd, jo * _L + iota, besti)
            return bestv, besti

        bestv, besti = lax.fori_loop(
            0, _B // _L, chunk,
            (jnp.full((_L,), -2.0, jnp.float32), jnp.zeros((_L,), jnp.int32)))

        m = jnp.max(bestv)
        idx = jnp.min(jnp.where(bestv == m, besti, jnp.int32(1 << 30)))
        sel = iota == r
        mvec = jnp.where(sel, jnp.full((_L,), m), mvec)
        ivec = jnp.where(sel, jnp.full((_L,), idx), ivec)

    res = jnp.where(mvec > -1.0, ivec, fb_v[...])
    res_v[...] = res

    @pl.when(is_pos)
    def _():
        pltpu.sync_copy(res_v, pos_hbm.at[pl.ds(row0, _RPW)])

    @pl.when(jnp.logical_not(is_pos))
    def _():
        pltpu.sync_copy(res_v, neg_hbm.at[pl.ds(row0, _RPW)])


@functools.cache
def _sc_select():
    mesh = plsc.VectorSubcoreMesh(core_axis_name="c", subcore_axis_name="s")
    return pl.kernel(
        _sc_select_body,
        mesh=mesh,
        compiler_params=pltpu.CompilerParams(needs_layout_passes=False),
        out_type=[jax.ShapeDtypeStruct((_B,), jnp.int32),
                  jax.ShapeDtypeStruct((_B,), jnp.int32)],
        scratch_types=[pltpu.VMEM((_B,), jnp.int32),
                       pltpu.VMEM((_RPW * _B,), jnp.float32),
                       pltpu.VMEM((_RPW,), jnp.int32),
                       pltpu.VMEM((_L,), jnp.int32)],
    )


def _worker_g_layout():
    import numpy as np
    g = np.empty((2 * _NS, _RPW * _B), np.float32)
    for w in range(_NS):
        g[w] = _GP[w * _RPW:(w + 1) * _RPW, :].reshape(-1)
        g[_NS + w] = _GN[w * _RPW:(w + 1) * _RPW, :].reshape(-1)
    return g


_GW = _worker_g_layout()


def _gram_kernel(f_ref, gram_ref, aux_ref):
    f = f_ref[...]
    gram_ref[...] = lax.dot_general(f, f, (((1,), (1,)), ((), ())),
                                    preferred_element_type=jnp.float32)
    sq = jnp.sum(f * f, axis=1, keepdims=True)
    rs = jnp.sum(f, axis=1, keepdims=True)
    aux_ref[...] = jnp.concatenate([sq, rs], axis=1)


def _loss_kernel(gram_ref, aux_ref, pidx_ref, nidx_ref, out_ref):
    iota_j = lax.broadcasted_iota(jnp.int32, (_B, _B), 1)
    P = (iota_j == pidx_ref[...]).astype(jnp.float32)
    N = (iota_j == nidx_ref[...]).astype(jnp.float32)

    gram = gram_ref[...]
    aux = aux_ref[...]
    dotp = lax.dot_general(P, gram, (((1,), (0,)), ((), ())),
                           preferred_element_type=jnp.float32)
    dotn = lax.dot_general(N, gram, (((1,), (0,)), ((), ())),
                           preferred_element_type=jnp.float32)
    auxp = lax.dot_general(P, aux, (((1,), (0,)), ((), ())),
                           preferred_element_type=jnp.float32)
    auxn = lax.dot_general(N, aux, (((1,), (0,)), ((), ())),
                           preferred_element_type=jnp.float32)

    sq_row = jnp.transpose(aux[:, 0:1])
    rs_row = jnp.transpose(aux[:, 1:2])
    const = float(_D) * _EPS * _EPS

    sqp = sq_row + auxp[:, 0:1] - 2.0 * dotp \
        + 2.0 * _EPS * (rs_row - auxp[:, 1:2]) + const
    sqn = sq_row + auxn[:, 0:1] - 2.0 * dotn \
        + 2.0 * _EPS * (rs_row - auxn[:, 1:2]) + const
    dp = jnp.sqrt(jnp.maximum(sqp, 1e-12))
    dn = jnp.sqrt(jnp.maximum(sqn, 1e-12))
    loss = jnp.sum(jnp.maximum(dp - dn + _MARGIN, 0.0),
                   keepdims=True) * (1.0 / (_B * _B))
    out_ref[...] = loss


@jax.jit
def kernel(features, labels):
    gw = jnp.asarray(_GW)
    fb = jnp.asarray(_FB)

    pos_idx, neg_idx = _sc_select()(labels, gw, fb)

    gram, aux = pl.pallas_call(
        _gram_kernel,
        out_shape=[jax.ShapeDtypeStruct((_B, _B), jnp.float32),
                   jax.ShapeDtypeStruct((_B, 2), jnp.float32)],
    )(features)

    out = pl.pallas_call(
        _loss_kernel,
        out_shape=jax.ShapeDtypeStruct((1, 1), jnp.float32),
    )(gram, aux, pos_idx.reshape(_B, 1), neg_idx.reshape(_B, 1))
    return out.reshape(())


# speedup vs baseline: 1.1209x; 1.0009x over previous
import functools

import jax
import jax.numpy as jnp
from jax import lax
from jax.experimental import pallas as pl
from jax.experimental.pallas import tpu as pltpu
from jax.experimental.pallas import tpu_sc as plsc

_MARGIN = 0.3
_EPS = 1e-6
_B = 256
_D = 2048
_L = 16
_NC = 2
_NS = 16
_RPW = _B // _L


def _threefry2x32(k0, k1, c0, c1):
    import numpy as np

    def rotl(x, r):
        return ((x << np.uint32(r)) | (x >> np.uint32(32 - r))).astype(np.uint32)

    ks0, ks1 = np.uint32(k0), np.uint32(k1)
    ks2 = np.uint32(ks0 ^ ks1 ^ np.uint32(0x1BD11BDA))
    x0 = (np.asarray(c0, np.uint32) + ks0).astype(np.uint32)
    x1 = (np.asarray(c1, np.uint32) + ks1).astype(np.uint32)
    rot = [13, 15, 26, 6, 17, 29, 16, 24]
    inject = [(ks1, ks2), (ks2, ks0), (ks0, ks1), (ks1, ks2), (ks2, ks0)]
    for block in range(5):
        for r in (rot[:4] if block % 2 == 0 else rot[4:]):
            x0 = (x0 + x1).astype(np.uint32)
            x1 = rotl(x1, r)
            x1 = (x1 ^ x0).astype(np.uint32)
        a, b = inject[block]
        x0 = (x0 + a).astype(np.uint32)
        x1 = (x1 + b + np.uint32(block + 1)).astype(np.uint32)
    return x0, x1


def _selection_constants():
    import numpy as np

    def bits(kpair, n):
        x0, x1 = _threefry2x32(kpair[0], kpair[1],
                               np.zeros(n, np.uint32), np.arange(n, dtype=np.uint32))
        return (x0 ^ x1).astype(np.uint32)

    def uniform(kpair, n):
        b = bits(kpair, n)
        return (((b >> np.uint32(9)) | np.uint32(0x3F800000))
                .view(np.float32) - np.float32(1.0))

    s0, s1 = _threefry2x32(0, 42, np.zeros(3, np.uint32),
                           np.arange(3, dtype=np.uint32))
    gp = uniform((s0[0], s1[0]), _B * _B).reshape(_B, _B)
    gn = uniform((s0[1], s1[1]), _B * _B).reshape(_B, _B)
    fb = np.array([
        49, 93, 107, 176, 77, 114, 51, 105, 130, 195, 217, 87, 120, 11, 158,
        226, 12, 194, 253, 69, 5, 212, 247, 10, 133, 85, 245, 148, 151, 21,
        85, 102, 134, 124, 40, 8, 221, 89, 168, 108, 46, 154, 166, 72, 79,
        247, 19, 10, 114, 97, 15, 77, 12, 147, 251, 16, 62, 79, 122, 230,
        220, 73, 255, 234, 10, 7, 68, 201, 10, 163, 63, 99, 86, 238, 223,
        225, 123, 53, 46, 45, 17, 243, 96, 79, 210, 106, 69, 109, 158, 13,
        165, 189, 155, 144, 61, 196, 34, 114, 177, 153, 81, 100, 47, 114,
        19, 27, 193, 146, 144, 255, 55, 68, 208, 64, 149, 244, 2, 101, 151,
        122, 40, 107, 24, 8, 127, 37, 24, 18, 27, 221, 33, 238, 66, 162,
        123, 151, 243, 149, 67, 177, 201, 202, 34, 250, 251, 7, 154, 16,
        222, 33, 75, 28, 120, 33, 232, 157, 170, 82, 124, 216, 91, 239, 147,
        162, 29, 60, 239, 153, 41, 106, 188, 95, 157, 76, 181, 70, 114, 71,
        216, 227, 9, 186, 77, 246, 94, 27, 111, 167, 100, 59, 134, 203, 246,
        241, 223, 60, 189, 156, 212, 129, 33, 111, 228, 52, 117, 145, 180,
        135, 69, 31, 101, 15, 250, 169, 151, 41, 231, 83, 93, 50, 9, 161,
        238, 221, 224, 3, 65, 155, 5, 194, 84, 70, 221, 114, 10, 141, 161,
        44, 10, 79, 119, 91, 181, 181, 59, 237, 86, 17, 51, 247, 139, 222,
        214, 6, 4, 3], dtype=np.int32)
    return gp, gn, fb


_GP, _GN, _FB = _selection_constants()


def _sc_select_body(labels_hbm, g_hbm, fb_hbm, pos_hbm, neg_hbm,
                    labels_v, g_v, fb_v, res_v):
    wid = lax.axis_index("s") * _NC + lax.axis_index("c")
    is_pos = wid < _NS
    row0 = jnp.where(is_pos, wid, wid - _NS) * _RPW

    pltpu.sync_copy(labels_hbm, labels_v)
    pltpu.sync_copy(g_hbm.at[wid], g_v)
    pltpu.sync_copy(fb_hbm.at[pl.ds(row0, _RPW)], fb_v)

    flip = jnp.full((_L,), (wid >= _NS).astype(jnp.int32))
    iota = lax.iota(jnp.int32, _L)
    mvec = jnp.full((_L,), -2.0, jnp.float32)
    ivec = jnp.zeros((_L,), jnp.int32)

    for r in range(_RPW):
        myl = plsc.load_gather(
            labels_v, [jnp.full((_L,), row0 + r, jnp.int32)])

        bestv = jnp.full((_L,), -2.0, jnp.float32)
        besti = jnp.zeros((_L,), jnp.int32)
        for jo in range(_B // _L):
            lab_c = labels_v[pl.ds(jo * _L, _L)]
            g_c = g_v[pl.ds(jo * _L + r * _B, _L)]
            same = (lab_c == myl).astype(jnp.int32)
            mg = jnp.where((same ^ flip) != 0, g_c, -1.0)
            upd = mg > bestv
            bestv = jnp.where(upd, mg, bestv)
            besti = jnp.where(upd, jo * _L + iota, besti)

        m = jnp.max(bestv)
        idx = jnp.min(jnp.where(bestv == m, besti, jnp.int32(1 << 30)))
        sel = iota == r
        mvec = jnp.where(sel, jnp.full((_L,), m), mvec)
        ivec = jnp.where(sel, jnp.full((_L,), idx), ivec)

    res = jnp.where(mvec > -1.0, ivec, fb_v[...])
    res_v[...] = res

    @pl.when(is_pos)
    def _():
        pltpu.sync_copy(res_v, pos_hbm.at[pl.ds(row0, _RPW)])

    @pl.when(jnp.logical_not(is_pos))
    def _():
        pltpu.sync_copy(res_v, neg_hbm.at[pl.ds(row0, _RPW)])


@functools.cache
def _sc_select():
    mesh = plsc.VectorSubcoreMesh(core_axis_name="c", subcore_axis_name="s")
    return pl.kernel(
        _sc_select_body,
        mesh=mesh,
        compiler_params=pltpu.CompilerParams(needs_layout_passes=False),
        out_type=[jax.ShapeDtypeStruct((_B,), jnp.int32),
                  jax.ShapeDtypeStruct((_B,), jnp.int32)],
        scratch_types=[pltpu.VMEM((_B,), jnp.int32),
                       pltpu.VMEM((_RPW * _B,), jnp.float32),
                       pltpu.VMEM((_RPW,), jnp.int32),
                       pltpu.VMEM((_L,), jnp.int32)],
    )


def _worker_g_layout():
    import numpy as np
    g = np.empty((2 * _NS, _RPW * _B), np.float32)
    for w in range(_NS):
        g[w] = _GP[w * _RPW:(w + 1) * _RPW, :].reshape(-1)
        g[_NS + w] = _GN[w * _RPW:(w + 1) * _RPW, :].reshape(-1)
    return g


_GW = _worker_g_layout()


def _gram_kernel(f_ref, gram_ref, aux_ref):
    f = f_ref[...]
    gram_ref[...] = lax.dot_general(f, f, (((1,), (1,)), ((), ())),
                                    preferred_element_type=jnp.float32)
    sq = jnp.sum(f * f, axis=1, keepdims=True)
    rs = jnp.sum(f, axis=1, keepdims=True)
    aux_ref[...] = jnp.concatenate([sq, rs], axis=1)


def _loss_kernel(gram_ref, aux_ref, pidx_ref, nidx_ref, out_ref):
    iota_j = lax.broadcasted_iota(jnp.int32, (_B, _B), 1)
    P = (iota_j == pidx_ref[...]).astype(jnp.float32)
    N = (iota_j == nidx_ref[...]).astype(jnp.float32)

    gram = gram_ref[...]
    aux = aux_ref[...]
    dotp = lax.dot_general(P, gram, (((1,), (0,)), ((), ())),
                           preferred_element_type=jnp.float32)
    dotn = lax.dot_general(N, gram, (((1,), (0,)), ((), ())),
                           preferred_element_type=jnp.float32)
    auxp = lax.dot_general(P, aux, (((1,), (0,)), ((), ())),
                           preferred_element_type=jnp.float32)
    auxn = lax.dot_general(N, aux, (((1,), (0,)), ((), ())),
                           preferred_element_type=jnp.float32)

    sq_row = jnp.transpose(aux[:, 0:1])
    rs_row = jnp.transpose(aux[:, 1:2])
    const = float(_D) * _EPS * _EPS

    sqp = sq_row + auxp[:, 0:1] - 2.0 * dotp \
        + 2.0 * _EPS * (rs_row - auxp[:, 1:2]) + const
    sqn = sq_row + auxn[:, 0:1] - 2.0 * dotn \
        + 2.0 * _EPS * (rs_row - auxn[:, 1:2]) + const
    dp = jnp.sqrt(jnp.maximum(sqp, 1e-12))
    dn = jnp.sqrt(jnp.maximum(sqn, 1e-12))
    loss = jnp.sum(jnp.maximum(dp - dn + _MARGIN, 0.0),
                   keepdims=True) * (1.0 / (_B * _B))
    out_ref[...] = loss


@jax.jit
def kernel(features, labels):
    gw = jnp.asarray(_GW)
    fb = jnp.asarray(_FB)

    pos_idx, neg_idx = _sc_select()(labels, gw, fb)

    gram, aux = pl.pallas_call(
        _gram_kernel,
        out_shape=[jax.ShapeDtypeStruct((_B, _B), jnp.float32),
                   jax.ShapeDtypeStruct((_B, 2), jnp.float32)],
    )(features)

    out = pl.pallas_call(
        _loss_kernel,
        out_shape=jax.ShapeDtypeStruct((1, 1), jnp.float32),
    )(gram, aux, pos_idx.reshape(_B, 1), neg_idx.reshape(_B, 1))
    return out.reshape(())


# speedup vs baseline: 1.1635x; 1.0380x over previous
import functools

import jax
import jax.numpy as jnp
from jax import lax
from jax.experimental import pallas as pl
from jax.experimental.pallas import tpu as pltpu
from jax.experimental.pallas import tpu_sc as plsc

_MARGIN = 0.3
_EPS = 1e-6
_B = 256
_D = 2048
_L = 16
_NC = 2
_NS = 16
_RPW = _B // _L


def _threefry2x32(k0, k1, c0, c1):
    import numpy as np

    def rotl(x, r):
        return ((x << np.uint32(r)) | (x >> np.uint32(32 - r))).astype(np.uint32)

    ks0, ks1 = np.uint32(k0), np.uint32(k1)
    ks2 = np.uint32(ks0 ^ ks1 ^ np.uint32(0x1BD11BDA))
    x0 = (np.asarray(c0, np.uint32) + ks0).astype(np.uint32)
    x1 = (np.asarray(c1, np.uint32) + ks1).astype(np.uint32)
    rot = [13, 15, 26, 6, 17, 29, 16, 24]
    inject = [(ks1, ks2), (ks2, ks0), (ks0, ks1), (ks1, ks2), (ks2, ks0)]
    for block in range(5):
        for r in (rot[:4] if block % 2 == 0 else rot[4:]):
            x0 = (x0 + x1).astype(np.uint32)
            x1 = rotl(x1, r)
            x1 = (x1 ^ x0).astype(np.uint32)
        a, b = inject[block]
        x0 = (x0 + a).astype(np.uint32)
        x1 = (x1 + b + np.uint32(block + 1)).astype(np.uint32)
    return x0, x1


def _selection_constants():
    import numpy as np

    def bits(kpair, n):
        x0, x1 = _threefry2x32(kpair[0], kpair[1],
                               np.zeros(n, np.uint32), np.arange(n, dtype=np.uint32))
        return (x0 ^ x1).astype(np.uint32)

    def uniform(kpair, n):
        b = bits(kpair, n)
        return (((b >> np.uint32(9)) | np.uint32(0x3F800000))
                .view(np.float32) - np.float32(1.0))

    s0, s1 = _threefry2x32(0, 42, np.zeros(3, np.uint32),
                           np.arange(3, dtype=np.uint32))
    gp = uniform((s0[0], s1[0]), _B * _B).reshape(_B, _B)
    gn = uniform((s0[1], s1[1]), _B * _B).reshape(_B, _B)
    fb = np.array([
        49, 93, 107, 176, 77, 114, 51, 105, 130, 195, 217, 87, 120, 11, 158,
        226, 12, 194, 253, 69, 5, 212, 247, 10, 133, 85, 245, 148, 151, 21,
        85, 102, 134, 124, 40, 8, 221, 89, 168, 108, 46, 154, 166, 72, 79,
        247, 19, 10, 114, 97, 15, 77, 12, 147, 251, 16, 62, 79, 122, 230,
        220, 73, 255, 234, 10, 7, 68, 201, 10, 163, 63, 99, 86, 238, 223,
        225, 123, 53, 46, 45, 17, 243, 96, 79, 210, 106, 69, 109, 158, 13,
        165, 189, 155, 144, 61, 196, 34, 114, 177, 153, 81, 100, 47, 114,
        19, 27, 193, 146, 144, 255, 55, 68, 208, 64, 149, 244, 2, 101, 151,
        122, 40, 107, 24, 8, 127, 37, 24, 18, 27, 221, 33, 238, 66, 162,
        123, 151, 243, 149, 67, 177, 201, 202, 34, 250, 251, 7, 154, 16,
        222, 33, 75, 28, 120, 33, 232, 157, 170, 82, 124, 216, 91, 239, 147,
        162, 29, 60, 239, 153, 41, 106, 188, 95, 157, 76, 181, 70, 114, 71,
        216, 227, 9, 186, 77, 246, 94, 27, 111, 167, 100, 59, 134, 203, 246,
        241, 223, 60, 189, 156, 212, 129, 33, 111, 228, 52, 117, 145, 180,
        135, 69, 31, 101, 15, 250, 169, 151, 41, 231, 83, 93, 50, 9, 161,
        238, 221, 224, 3, 65, 155, 5, 194, 84, 70, 221, 114, 10, 141, 161,
        44, 10, 79, 119, 91, 181, 181, 59, 237, 86, 17, 51, 247, 139, 222,
        214, 6, 4, 3], dtype=np.int32)
    return gp, gn, fb


_GP, _GN, _FB = _selection_constants()


def _sc_select_body(labels_hbm, g_hbm, fb_hbm, pos_hbm, neg_hbm,
                    labels_v, g_v, fb_v, res_v, s_lab, s_glo, s_ghi, s_fb):
    wid = lax.axis_index("s") * _NC + lax.axis_index("c")
    is_pos = wid < _NS
    row0 = jnp.where(is_pos, wid, wid - _NS) * _RPW

    half = _RPW * _B // 2
    c_lab = pltpu.async_copy(labels_hbm, labels_v, s_lab)
    c_glo = pltpu.async_copy(g_hbm.at[wid, pl.ds(0, half)],
                             g_v.at[pl.ds(0, half)], s_glo)
    c_ghi = pltpu.async_copy(g_hbm.at[wid, pl.ds(half, half)],
                             g_v.at[pl.ds(half, half)], s_ghi)
    c_fb = pltpu.async_copy(fb_hbm.at[pl.ds(row0, _RPW)], fb_v, s_fb)

    flip = jnp.full((_L,), (wid >= _NS).astype(jnp.int32))
    iota = lax.iota(jnp.int32, _L)
    mvec = jnp.full((_L,), -2.0, jnp.float32)
    ivec = jnp.zeros((_L,), jnp.int32)

    c_lab.wait()
    c_glo.wait()
    for r in range(_RPW):
        if r == _RPW // 2:
            c_ghi.wait()
        myl = plsc.load_gather(
            labels_v, [jnp.full((_L,), row0 + r, jnp.int32)])

        bestv = jnp.full((_L,), -2.0, jnp.float32)
        besti = jnp.zeros((_L,), jnp.int32)
        for jo in range(_B // _L):
            lab_c = labels_v[pl.ds(jo * _L, _L)]
            g_c = g_v[pl.ds(jo * _L + r * _B, _L)]
            same = (lab_c == myl).astype(jnp.int32)
            mg = jnp.where((same ^ flip) != 0, g_c, -1.0)
            upd = mg > bestv
            bestv = jnp.where(upd, mg, bestv)
            besti = jnp.where(upd, jo * _L + iota, besti)

        m = jnp.max(bestv)
        idx = jnp.min(jnp.where(bestv == m, besti, jnp.int32(1 << 30)))
        sel = iota == r
        mvec = jnp.where(sel, jnp.full((_L,), m), mvec)
        ivec = jnp.where(sel, jnp.full((_L,), idx), ivec)

    c_fb.wait()
    res = jnp.where(mvec > -1.0, ivec, fb_v[...])
    res_v[...] = res

    @pl.when(is_pos)
    def _():
        pltpu.sync_copy(res_v, pos_hbm.at[pl.ds(row0, _RPW)])

    @pl.when(jnp.logical_not(is_pos))
    def _():
        pltpu.sync_copy(res_v, neg_hbm.at[pl.ds(row0, _RPW)])


@functools.cache
def _sc_select():
    mesh = plsc.VectorSubcoreMesh(core_axis_name="c", subcore_axis_name="s")
    return pl.kernel(
        _sc_select_body,
        mesh=mesh,
        compiler_params=pltpu.CompilerParams(needs_layout_passes=False),
        out_type=[jax.ShapeDtypeStruct((_B,), jnp.int32),
                  jax.ShapeDtypeStruct((_B,), jnp.int32)],
        scratch_types=[pltpu.VMEM((_B,), jnp.int32),
                       pltpu.VMEM((_RPW * _B,), jnp.float32),
                       pltpu.VMEM((_RPW,), jnp.int32),
                       pltpu.VMEM((_L,), jnp.int32),
                       pltpu.SemaphoreType.DMA,
                       pltpu.SemaphoreType.DMA,
                       pltpu.SemaphoreType.DMA,
                       pltpu.SemaphoreType.DMA],
    )


def _worker_g_layout():
    import numpy as np
    g = np.empty((2 * _NS, _RPW * _B), np.float32)
    for w in range(_NS):
        g[w] = _GP[w * _RPW:(w + 1) * _RPW, :].reshape(-1)
        g[_NS + w] = _GN[w * _RPW:(w + 1) * _RPW, :].reshape(-1)
    return g


_GW = _worker_g_layout()


def _gram_kernel(f_ref, gram_ref, aux_ref):
    f = f_ref[...]
    gram_ref[...] = lax.dot_general(f, f, (((1,), (1,)), ((), ())),
                                    preferred_element_type=jnp.float32)
    sq = jnp.sum(f * f, axis=1, keepdims=True)
    rs = jnp.sum(f, axis=1, keepdims=True)
    aux_ref[...] = jnp.concatenate([sq, rs], axis=1)


def _loss_kernel(gram_ref, aux_ref, pidx_ref, nidx_ref, out_ref):
    iota_j = lax.broadcasted_iota(jnp.int32, (_B, _B), 1)
    P = (iota_j == pidx_ref[...]).astype(jnp.float32)
    N = (iota_j == nidx_ref[...]).astype(jnp.float32)

    gram = gram_ref[...]
    aux = aux_ref[...]
    dotp = lax.dot_general(P, gram, (((1,), (0,)), ((), ())),
                           preferred_element_type=jnp.float32)
    dotn = lax.dot_general(N, gram, (((1,), (0,)), ((), ())),
                           preferred_element_type=jnp.float32)
    auxp = lax.dot_general(P, aux, (((1,), (0,)), ((), ())),
                           preferred_element_type=jnp.float32)
    auxn = lax.dot_general(N, aux, (((1,), (0,)), ((), ())),
                           preferred_element_type=jnp.float32)

    sq_row = jnp.transpose(aux[:, 0:1])
    rs_row = jnp.transpose(aux[:, 1:2])
    const = float(_D) * _EPS * _EPS

    sqp = sq_row + auxp[:, 0:1] - 2.0 * dotp \
        + 2.0 * _EPS * (rs_row - auxp[:, 1:2]) + const
    sqn = sq_row + auxn[:, 0:1] - 2.0 * dotn \
        + 2.0 * _EPS * (rs_row - auxn[:, 1:2]) + const
    dp = jnp.sqrt(jnp.maximum(sqp, 1e-12))
    dn = jnp.sqrt(jnp.maximum(sqn, 1e-12))
    loss = jnp.sum(jnp.maximum(dp - dn + _MARGIN, 0.0),
                   keepdims=True) * (1.0 / (_B * _B))
    out_ref[...] = loss


@jax.jit
def kernel(features, labels):
    gw = jnp.asarray(_GW)
    fb = jnp.asarray(_FB)

    pos_idx, neg_idx = _sc_select()(labels, gw, fb)

    gram, aux = pl.pallas_call(
        _gram_kernel,
        out_shape=[jax.ShapeDtypeStruct((_B, _B), jnp.float32),
                   jax.ShapeDtypeStruct((_B, 2), jnp.float32)],
    )(features)

    out = pl.pallas_call(
        _loss_kernel,
        out_shape=jax.ShapeDtypeStruct((1, 1), jnp.float32),
    )(gram, aux, pos_idx.reshape(_B, 1), neg_idx.reshape(_B, 1))
    return out.reshape(())
